# baseline (speedup 1.0000x reference)
import jax
import jax.numpy as jnp
from jax.experimental import pallas as pl
from jax.experimental.pallas import tpu as pltpu

D_MODEL = 1024
NUM_HEADS = 16
D_K = 64
THETA = 10000.0
EPS = 1e-8
BQ = 1024
LOG2E = 1.4426950408889634


def _attn_kernel(x_ref, w_ref, wo_ref, cos_ref, sin_ref, g_ref, o_ref,
                 qkv_ref, qn_ref, kn_ref, vt_ref, at_ref):
    seq = x_ref.shape[1]
    hpid = pl.program_id(1)
    cos = cos_ref[...]
    sin = sin_ref[...]

    qkv_ref[...] = jnp.dot(x_ref[0], w_ref[0],
                           preferred_element_type=jnp.float32)

    ri = jax.lax.broadcasted_iota(jnp.int32, (128, 128), 0)
    ci = jax.lax.broadcasted_iota(jnp.int32, (128, 128), 1)
    bd_ones = jnp.where((ri // D_K) == (ci // D_K), 1.0, 0.0)

    def rope_norm(xb):
        sw = jnp.concatenate(
            [xb[:, 32:64], xb[:, 0:32], xb[:, 96:128], xb[:, 64:96]], axis=1)
        r = cos * xb + sin * sw
        ssb = jnp.dot(r * r, bd_ones, preferred_element_type=jnp.float32)
        return r / (jnp.sqrt(ssb) + EPS)

    kn_ref[...] = rope_norm(qkv_ref[:, 128:256])
    qn_ref[...] = rope_norm(qkv_ref[:, 0:128]) * g_ref[0]

    eye = jnp.where(ri == ci, 1.0, 0.0).astype(jnp.bfloat16)
    vt_all = jax.lax.dot_general(
        eye, qkv_ref[:, 256:384].astype(jnp.bfloat16),
        (((1,), (1,)), ((), ())),
        preferred_element_type=jnp.float32).astype(jnp.bfloat16)
    vt_ref[0, 0:64, :] = vt_all[0:64]
    vt_ref[1, 0:64, :] = vt_all[64:128]
    ones_rows = jnp.ones((8, seq), jnp.bfloat16)
    vt_ref[0, 64:72, :] = ones_rows
    vt_ref[1, 64:72, :] = ones_rows

    rows_d = jax.lax.broadcasted_iota(jnp.int32, (BQ, BQ), 0)
    cols_d = jax.lax.broadcasted_iota(jnp.int32, (BQ, BQ), 1)
    dmask_t = rows_d <= cols_d

    for qi in range(seq // BQ):
        base = qi * BQ
        for s in range(2):
            lo, hi = s * D_K, (s + 1) * D_K
            qb = qn_ref[base:base + BQ, lo:hi]
            sc_d = jax.lax.dot_general(
                kn_ref[base:base + BQ, lo:hi], qb, (((1,), (1,)), ((), ())),
                preferred_element_type=jnp.float32)
            e_d = jnp.where(dmask_t, jnp.exp2(sc_d), 0.0)
            acc = jax.lax.dot_general(
                vt_ref[s, :, base:base + BQ], e_d.astype(jnp.bfloat16),
                (((1,), (0,)), ((), ())),
                preferred_element_type=jnp.float32)
            if qi > 0:
                sc_h = jax.lax.dot_general(
                    kn_ref[0:base, lo:hi], qb, (((1,), (1,)), ((), ())),
                    preferred_element_type=jnp.float32)
                e_h = jnp.exp2(sc_h)
                acc = acc + jax.lax.dot_general(
                    vt_ref[s, :, 0:base], e_h.astype(jnp.bfloat16),
                    (((1,), (0,)), ((), ())),
                    preferred_element_type=jnp.float32)
            at_ref[lo:hi, base:base + BQ] = (
                acc[0:64] * (1.0 / acc[64:65])).astype(jnp.bfloat16)

    first = hpid == 0
    wo = wo_ref[0]
    for tc in range(4):
        t0 = tc * (seq // 4)
        t1 = t0 + seq // 4
        pc = jax.lax.dot_general(
            at_ref[:, t0:t1], wo, (((0,), (0,)), ((), ())),
            preferred_element_type=jnp.float32)
        o_ref[0, t0:t1, :] = jnp.where(first, pc, o_ref[0, t0:t1, :] + pc)


def _fused(x, w_hp, wo_hp, cos, sin, garr):
    b, n, _ = x.shape
    hpairs = NUM_HEADS // 2
    return pl.pallas_call(
        _attn_kernel,
        grid=(b, hpairs),
        in_specs=[
            pl.BlockSpec((1, n, D_MODEL), lambda bi, hp: (bi, 0, 0)),
            pl.BlockSpec((1, D_MODEL, 384), lambda bi, hp: (hp, 0, 0)),
            pl.BlockSpec((1, 128, D_MODEL), lambda bi, hp: (hp, 0, 0)),
            pl.BlockSpec((n, 128), lambda bi, hp: (0, 0)),
            pl.BlockSpec((n, 128), lambda bi, hp: (0, 0)),
            pl.BlockSpec((1, 1, 128), lambda bi, hp: (hp, 0, 0)),
        ],
        out_specs=pl.BlockSpec((1, n, D_MODEL), lambda bi, hp: (bi, 0, 0)),
        out_shape=jax.ShapeDtypeStruct((b, n, D_MODEL), jnp.float32),
        scratch_shapes=[
            pltpu.VMEM((n, 384), jnp.float32),
            pltpu.VMEM((n, 128), jnp.float32),
            pltpu.VMEM((n, 128), jnp.float32),
            pltpu.VMEM((2, 72, n), jnp.bfloat16),
            pltpu.VMEM((128, n), jnp.bfloat16),
        ],
        compiler_params=pltpu.CompilerParams(
            dimension_semantics=("arbitrary", "arbitrary"),
            vmem_limit_bytes=100 * 1024 * 1024,
        ),
    )(x, w_hp, wo_hp, cos, sin, garr)


def kernel(x, token_positions, W_QKV, W_O, qk_scale):
    b, n, d = x.shape

    def permute_half_split(w):
        return (w.reshape(NUM_HEADS, D_K // 2, 2, d)
                 .transpose(0, 2, 1, 3).reshape(d, d))

    w_q = permute_half_split(W_QKV[:D_MODEL])
    w_k = permute_half_split(W_QKV[D_MODEL:2 * D_MODEL])
    w_v = W_QKV[2 * D_MODEL:]
    hp = NUM_HEADS // 2
    w_hp = jnp.stack([
        jnp.concatenate([w_q[i * 128:(i + 1) * 128],
                         w_k[i * 128:(i + 1) * 128],
                         w_v[i * 128:(i + 1) * 128]], axis=0).T
        for i in range(hp)], axis=0)

    wo_hp = W_O.T.reshape(hp, 128, d).astype(jnp.bfloat16)

    pos = token_positions.astype(jnp.float32)
    inv_theta = THETA ** (-(2.0 * jnp.arange(D_K // 2, dtype=jnp.float32))
                          / D_K)
    ang = pos[:, None] * inv_theta[None, :]
    c32, s32 = jnp.cos(ang), jnp.sin(ang)
    cos = jnp.tile(jnp.concatenate([c32, c32], axis=1), (1, 2))
    sin = jnp.tile(jnp.concatenate([-s32, s32], axis=1), (1, 2))

    garr = jnp.repeat(qk_scale * LOG2E, D_K).reshape(hp, 1, 128)

    return _fused(x, w_hp, wo_hp, cos, sin, garr)


# speedup vs baseline: 1.0602x; 1.0602x over previous
import jax
import jax.numpy as jnp
from jax.experimental import pallas as pl
from jax.experimental.pallas import tpu as pltpu

D_MODEL = 1024
NUM_HEADS = 16
D_K = 64
THETA = 10000.0
EPS = 1e-8
BQ = 512
LOG2E = 1.4426950408889634


def _attn_kernel(x_ref, w_ref, cos_ref, sin_ref, g_ref, o_ref,
                 qkv_ref, qn_ref, kn_ref, vt_ref):
    seq = x_ref.shape[1]
    cos = cos_ref[...]
    sin = sin_ref[...]

    qkv_ref[...] = jnp.dot(x_ref[0], w_ref[0],
                           preferred_element_type=jnp.float32)

    ri = jax.lax.broadcasted_iota(jnp.int32, (128, 128), 0)
    ci = jax.lax.broadcasted_iota(jnp.int32, (128, 128), 1)
    bd_ones = jnp.where((ri // D_K) == (ci // D_K), 1.0, 0.0)

    def rope_norm(xb):
        sw = jnp.concatenate(
            [xb[:, 32:64], xb[:, 0:32], xb[:, 96:128], xb[:, 64:96]], axis=1)
        r = cos * xb + sin * sw
        ssb = jnp.dot(r * r, bd_ones, preferred_element_type=jnp.float32)
        return r / (jnp.sqrt(ssb) + EPS)

    kn_ref[...] = rope_norm(qkv_ref[:, 128:256])
    qn_ref[...] = rope_norm(qkv_ref[:, 0:128]) * g_ref[0]

    eye = jnp.where(ri == ci, 1.0, 0.0).astype(jnp.bfloat16)
    vt_all = jax.lax.dot_general(
        eye, qkv_ref[:, 256:384].astype(jnp.bfloat16),
        (((1,), (1,)), ((), ())),
        preferred_element_type=jnp.float32).astype(jnp.bfloat16)
    vt_ref[0, 0:64, :] = vt_all[0:64]
    vt_ref[1, 0:64, :] = vt_all[64:128]
    ones_rows = jnp.ones((8, seq), jnp.bfloat16)
    vt_ref[0, 64:72, :] = ones_rows
    vt_ref[1, 64:72, :] = ones_rows

    rows_d = jax.lax.broadcasted_iota(jnp.int32, (BQ, BQ), 0)
    cols_d = jax.lax.broadcasted_iota(jnp.int32, (BQ, BQ), 1)
    dmask_t = rows_d <= cols_d

    for qi in range(seq // BQ):
        base = qi * BQ
        for s in range(2):
            lo, hi = s * D_K, (s + 1) * D_K
            qb = qn_ref[base:base + BQ, lo:hi]
            sc_d = jax.lax.dot_general(
                kn_ref[base:base + BQ, lo:hi], qb, (((1,), (1,)), ((), ())),
                preferred_element_type=jnp.float32)
            e_d = jnp.where(dmask_t, jnp.exp2(sc_d), 0.0)
            acc = jax.lax.dot_general(
                vt_ref[s, :, base:base + BQ], e_d.astype(jnp.bfloat16),
                (((1,), (0,)), ((), ())),
                preferred_element_type=jnp.float32)
            if qi > 0:
                sc_h = jax.lax.dot_general(
                    kn_ref[0:base, lo:hi], qb, (((1,), (1,)), ((), ())),
                    preferred_element_type=jnp.float32)
                e_h = jnp.exp2(sc_h)
                acc = acc + jax.lax.dot_general(
                    vt_ref[s, :, 0:base], e_h.astype(jnp.bfloat16),
                    (((1,), (0,)), ((), ())),
                    preferred_element_type=jnp.float32)
            o_ref[0, lo:hi, base:base + BQ] = (
                acc[0:64] * (1.0 / acc[64:65])).astype(jnp.bfloat16)


def _fused(x, w_hp, cos, sin, garr):
    b, n, _ = x.shape
    hpairs = NUM_HEADS // 2
    return pl.pallas_call(
        _attn_kernel,
        grid=(b, hpairs),
        in_specs=[
            pl.BlockSpec((1, n, D_MODEL), lambda bi, hp: (bi, 0, 0)),
            pl.BlockSpec((1, D_MODEL, 384), lambda bi, hp: (hp, 0, 0)),
            pl.BlockSpec((n, 128), lambda bi, hp: (0, 0)),
            pl.BlockSpec((n, 128), lambda bi, hp: (0, 0)),
            pl.BlockSpec((1, 1, 128), lambda bi, hp: (hp, 0, 0)),
        ],
        out_specs=pl.BlockSpec((1, 128, n), lambda bi, hp: (bi, hp, 0)),
        out_shape=jax.ShapeDtypeStruct((b, D_MODEL, n), jnp.bfloat16),
        scratch_shapes=[
            pltpu.VMEM((n, 384), jnp.float32),
            pltpu.VMEM((n, 128), jnp.float32),
            pltpu.VMEM((n, 128), jnp.float32),
            pltpu.VMEM((2, 72, n), jnp.bfloat16),
        ],
        compiler_params=pltpu.CompilerParams(
            dimension_semantics=("arbitrary", "arbitrary"),
            vmem_limit_bytes=100 * 1024 * 1024,
        ),
    )(x, w_hp, cos, sin, garr)


def _outproj_kernel(a_ref, w_ref, o_ref):
    o_ref[0] = jax.lax.dot_general(
        a_ref[0], w_ref[...].astype(jnp.bfloat16), (((0,), (0,)), ((), ())),
        preferred_element_type=jnp.float32)


def _outproj(attn_t, w_t, bt):
    b, d, n = attn_t.shape
    return pl.pallas_call(
        _outproj_kernel,
        grid=(b, n // bt),
        in_specs=[
            pl.BlockSpec((1, d, bt), lambda bi, j: (bi, 0, j)),
            pl.BlockSpec((d, D_MODEL), lambda bi, j: (0, 0)),
        ],
        out_specs=pl.BlockSpec((1, bt, D_MODEL), lambda bi, j: (bi, j, 0)),
        out_shape=jax.ShapeDtypeStruct((b, n, D_MODEL), jnp.float32),
        compiler_params=pltpu.CompilerParams(
            dimension_semantics=("arbitrary", "arbitrary"),
            vmem_limit_bytes=100 * 1024 * 1024,
        ),
    )(attn_t, w_t)


def kernel(x, token_positions, W_QKV, W_O, qk_scale):
    b, n, d = x.shape

    def permute_half_split(w):
        return (w.reshape(NUM_HEADS, D_K // 2, 2, d)
                 .transpose(0, 2, 1, 3).reshape(d, d))

    w_q = permute_half_split(W_QKV[:D_MODEL])
    w_k = permute_half_split(W_QKV[D_MODEL:2 * D_MODEL])
    w_v = W_QKV[2 * D_MODEL:]
    hp = NUM_HEADS // 2
    w_hp = jnp.stack([
        jnp.concatenate([w_q[i * 128:(i + 1) * 128],
                         w_k[i * 128:(i + 1) * 128],
                         w_v[i * 128:(i + 1) * 128]], axis=0).T
        for i in range(hp)], axis=0)

    pos = token_positions.astype(jnp.float32)
    inv_theta = THETA ** (-(2.0 * jnp.arange(D_K // 2, dtype=jnp.float32))
                          / D_K)
    ang = pos[:, None] * inv_theta[None, :]
    c32, s32 = jnp.cos(ang), jnp.sin(ang)
    cos = jnp.tile(jnp.concatenate([c32, c32], axis=1), (1, 2))
    sin = jnp.tile(jnp.concatenate([-s32, s32], axis=1), (1, 2))

    garr = jnp.repeat(qk_scale * LOG2E, D_K).reshape(hp, 1, 128)

    attn_t = _fused(x, w_hp, cos, sin, garr)
    return _outproj(attn_t, W_O.T, 1024)
